# baseline (speedup 1.0000x reference)
import jax
import jax.numpy as jnp
from jax import lax
from jax.experimental import pallas as pl
from jax.experimental.pallas import tpu as pltpu
from jax.experimental.pallas import tpu_sc as plsc

_B = 16384
_D = 32
_NC = 2
_NS = 16
_NW = _NC * _NS
_BPW = _B // _NW
_NG = _BPW // 16
_HP = 16
_FPW = _BPW * _HP
_ROWS128 = _B * _HP // 128
_RPB = 128 // _HP
_TCBLK = 512


_PASS = 128
_NPASS = _BPW // _PASS


def _gather_body(u_hbm, i_hbm, U_hbm, V_hbm, bu_hbm, bi_hbm,
                 prod_hbm, bsum_hbm,
                 u_idx, i_idx, u_rows, v_rows, prod, bu_rows, bi_rows,
                 sem_u0, sem_u1, sem_v0, sem_v1, sem_b):
    sid = lax.axis_index("s")
    wid = sid * _NC + lax.axis_index("c")
    sem_u = (sem_u0, sem_u1)
    sem_v = (sem_v0, sem_v1)

    pltpu.sync_copy(u_hbm.at[wid], u_idx)
    pltpu.sync_copy(i_hbm.at[wid], i_idx)

    bias_copies = []
    for c in range(4):
        sl = pl.ds(c * 128, 128)
        bias_copies.append(pltpu.async_copy(bu_hbm.at[u_idx.at[sl]],
                                            bu_rows.at[sl], sem_b))
        bias_copies.append(pltpu.async_copy(bi_hbm.at[i_idx.at[sl]],
                                            bi_rows.at[sl], sem_b))

    def fire(p, buf):
        def enq(k, carry):
            uvec = u_idx[pl.ds(p * _PASS + k * 16, 16)]
            ivec = i_idx[pl.ds(p * _PASS + k * 16, 16)]
            for r in range(16):
                b = k * 16 + r
                pltpu.async_copy(U_hbm.at[uvec[r]], u_rows.at[buf, b],
                                 sem_u[buf])
                pltpu.async_copy(V_hbm.at[ivec[r]], v_rows.at[buf, b],
                                 sem_v[buf])
            return carry
        lax.fori_loop(0, _PASS // 16, enq, 0)

    fire(0, 0)
    for p in range(_NPASS):
        buf = p % 2
        if p + 1 < _NPASS:
            fire(p + 1, (p + 1) % 2)
        pltpu.make_async_copy(U_hbm.at[pl.ds(0, _PASS)], u_rows.at[buf],
                              sem_u[buf]).wait()
        pltpu.make_async_copy(V_hbm.at[pl.ds(0, _PASS)], v_rows.at[buf],
                              sem_v[buf]).wait()

        def pbody(k, carry, buf=buf):
            p16 = (u_rows[buf, k, pl.ds(0, 16)] * v_rows[buf, k, pl.ds(0, 16)]
                   + u_rows[buf, k, pl.ds(16, 16)]
                   * v_rows[buf, k, pl.ds(16, 16)])
            prod[pl.ds(k * _HP, 16)] = p16
            return carry
        lax.fori_loop(0, _PASS, pbody, 0)
        pltpu.sync_copy(prod, prod_hbm.at[pl.ds(wid * _FPW + p * _PASS * _HP,
                                                _PASS * _HP)])

    for cp in bias_copies:
        cp.wait()

    def bbody(t, carry):
        sl = pl.ds(t * 16, 16)
        bu_rows[sl] = bu_rows[sl] + bi_rows[sl]
        return carry
    lax.fori_loop(0, _BPW // 16, bbody, 0)

    pltpu.sync_copy(bu_rows, bsum_hbm.at[pl.ds(wid * _BPW, _BPW)])


def _dot_body(mu_ref, p_ref, bsum_ref, o_ref):
    col = lax.broadcasted_iota(jnp.int32, (128, _RPB), 0)
    grp = lax.broadcasted_iota(jnp.int32, (128, _RPB), 1)
    sel = (col // _HP == grp).astype(jnp.float32)
    s = jax.lax.dot_general(p_ref[...], sel, (((1,), (0,)), ((), ())),
                            preferred_element_type=jnp.float32)
    o_ref[...] = s + bsum_ref[...] + mu_ref[0]


def kernel(u, i, U, V, bu, bi, mu):
    u2 = u.reshape(_NW, _BPW)
    i2 = i.reshape(_NW, _BPW)
    bu_flat = bu.reshape(-1)
    bi_flat = bi.reshape(-1)

    mesh = plsc.VectorSubcoreMesh(core_axis_name="c", subcore_axis_name="s",
                                  num_cores=_NC, num_subcores=_NS)
    gather = pl.kernel(
        _gather_body,
        out_type=(
            jax.ShapeDtypeStruct((_B * _HP,), jnp.float32),
            jax.ShapeDtypeStruct((_B,), jnp.float32),
        ),
        mesh=mesh,
        scratch_types=[
            pltpu.VMEM((_BPW,), jnp.int32),
            pltpu.VMEM((_BPW,), jnp.int32),
            pltpu.VMEM((2, _PASS, _D), jnp.float32),
            pltpu.VMEM((2, _PASS, _D), jnp.float32),
            pltpu.VMEM((_PASS * _HP,), jnp.float32),
            pltpu.VMEM((_BPW,), jnp.float32),
            pltpu.VMEM((_BPW,), jnp.float32),
            pltpu.SemaphoreType.DMA,
            pltpu.SemaphoreType.DMA,
            pltpu.SemaphoreType.DMA,
            pltpu.SemaphoreType.DMA,
            pltpu.SemaphoreType.DMA,
        ],
    )
    prod, bsum = gather(u2, i2, U, V, bu_flat, bi_flat)
    prod4 = prod.reshape(_ROWS128, 128)
    bsum4 = bsum.reshape(_ROWS128, _RPB)

    dot = pl.pallas_call(
        _dot_body,
        out_shape=jax.ShapeDtypeStruct((_ROWS128, _RPB), jnp.float32),
        grid=(_ROWS128 // _TCBLK,),
        in_specs=[
            pl.BlockSpec(memory_space=pltpu.SMEM),
            pl.BlockSpec((_TCBLK, 128), lambda g: (g, 0)),
            pl.BlockSpec((_TCBLK, _RPB), lambda g: (g, 0)),
        ],
        out_specs=pl.BlockSpec((_TCBLK, _RPB), lambda g: (g, 0)),
    )
    out4 = dot(mu, prod4, bsum4)
    return out4.reshape(_B)
